# Initial kernel scaffold; baseline (speedup 1.0000x reference)
import jax
import jax.numpy as jnp
from jax.experimental import pallas as pl


def kernel(x, edge_index, pseudo, batch, W1, R1, B1, W2, R2, B2, W3, R3, B3, W4, R4, B4, fcw, fcb):
    raise NotImplementedError("write your pallas kernel here")


# speedup vs baseline: 1.9987x; 1.9987x over previous
import functools

import jax
import jax.numpy as jnp
from jax import lax
from jax.experimental import pallas as pl
from jax.experimental.pallas import tpu as pltpu
from jax.experimental.pallas import tpu_sc as plsc

N = 50000
E = 800000
G = 16
F = 64
HF = 32
QF = 16
EPAD = 819200
NSC = 16
EW = EPAD // NSC
SUP = 1024
CH = 128
NSUB = SUP // CH
NCHUNK = EW // SUP
NACC = 50176
ZROWS = NACC // NSC
OROWS = N // NSC
BLK = 2000
NBLK = N // BLK


def _zero_z(z):
    def zrow(r, _):
        z[r, pl.ds(0, 16)] = jnp.zeros((16,), jnp.float32)
        return 0
    lax.fori_loop(0, SUP, zrow, 0)


def _zero_acc_slice(z, acc, s):
    zbase = s * ZROWS
    for j in range(3):
        pltpu.sync_copy(z, acc.at[pl.ds(zbase + j * SUP, SUP)])
    pltpu.sync_copy(z.at[pl.ds(0, ZROWS - 3 * SUP)],
                    acc.at[pl.ds(zbase + 3 * SUP, ZROWS - 3 * SUP)])


def _copy_out(acc, z, out_slice_fn, s):
    obase = s * ZROWS
    off = 0
    for nrows in (SUP, SUP, SUP, ZROWS - 3 * SUP):
        r0 = pl.multiple_of(obase + off, 8)
        pltpu.sync_copy(acc.at[pl.ds(r0, nrows)], z.at[pl.ds(0, nrows)])
        pltpu.sync_copy(z.at[pl.ds(0, nrows)], out_slice_fn(r0, nrows))
        off += nrows


def _build_sc_agg(want_deg):
    mesh = plsc.VectorSubcoreMesh(core_axis_name="c", subcore_axis_name="s")
    agg_t = jax.ShapeDtypeStruct((2, 2, NACC, QF), jnp.float32)
    outs = [agg_t, jax.ShapeDtypeStruct((NACC, QF), jnp.float32)] if want_deg else agg_t
    scratch = [
        pltpu.VMEM((SUP,), jnp.int32),
        pltpu.VMEM((NSUB, CH), jnp.int32),
        pltpu.VMEM((SUP + 16,), jnp.float32),
        pltpu.VMEM((SUP, 2 * QF), jnp.float32),
        pltpu.VMEM((SUP, QF), jnp.float32),
        pltpu.VMEM_SHARED((NACC, QF), jnp.float32),
        pltpu.SemaphoreType.DMA,
    ]
    if want_deg:
        scratch.append(pltpu.VMEM((CH, QF), jnp.float32))

    def body(tflat, src4, dst2, u_all, out_agg, *rest):
        if want_deg:
            out_deg, isrc, idst, uv, rows, z, acc, sem, ones = rest
        else:
            isrc, idst, uv, rows, z, acc, sem = rest
        c = lax.axis_index("c")
        s = lax.axis_index("s")
        ebase = s * EW

        def agg_phase(p):
            _zero_z(z)
            _zero_acc_slice(z, acc, s)
            plsc.subcore_barrier()
            soff = (c + 2 * p) * EPAD

            def chunk(k, _):
                base = pl.multiple_of(ebase + k * SUP, SUP)
                pltpu.sync_copy(src4.at[pl.ds(pl.multiple_of(soff + base, SUP), SUP)], isrc)
                pltpu.sync_copy(dst2.at[pl.ds(pl.multiple_of(base // CH, 8), NSUB)], idst)
                pltpu.sync_copy(u_all.at[pl.ds(base, SUP)], uv.at[pl.ds(0, SUP)])
                cps = [
                    pltpu.async_copy(tflat.at[isrc.at[pl.ds(j * CH, CH)]],
                                     rows.at[pl.ds(j * CH, CH)], sem)
                    for j in range(NSUB)
                ]
                for cp in cps:
                    cp.wait()

                def edge(e, _):
                    ue = jnp.full((16,), uv[pl.ds(e, 16)][0], jnp.float32)
                    y0 = rows[e, pl.ds(0, 16)]
                    yd = rows[e, pl.ds(16, 16)]
                    z[e, pl.ds(0, 16)] = y0 + ue * yd
                    return 0
                lax.fori_loop(0, SUP, edge, 0)

                for j in range(NSUB):
                    pltpu.sync_copy(z.at[pl.ds(j * CH, CH)],
                                    acc.at[idst.at[j]], add=True)
                return 0
            lax.fori_loop(0, NCHUNK, chunk, 0)
            plsc.subcore_barrier()
            _copy_out(acc, z, lambda r0, nr: out_agg.at[p, c, pl.ds(r0, nr)], s)

        agg_phase(0)
        agg_phase(1)

        if want_deg:
            @pl.when(c == 0)
            def _deg_phase():
                def orow(r, _):
                    ones[r, pl.ds(0, 16)] = jnp.ones((16,), jnp.float32)
                    return 0
                lax.fori_loop(0, CH, orow, 0)
                _zero_z(z)
                _zero_acc_slice(z, acc, s)
                plsc.subcore_barrier()

                def dchunk(k, _):
                    base = pl.multiple_of(ebase + k * SUP, SUP)
                    pltpu.sync_copy(dst2.at[pl.ds(pl.multiple_of(base // CH, 8), NSUB)], idst)
                    for j in range(NSUB):
                        pltpu.sync_copy(ones, acc.at[idst.at[j]], add=True)
                    return 0
                lax.fori_loop(0, NCHUNK, dchunk, 0)
                plsc.subcore_barrier()
                _copy_out(acc, z, lambda r0, nr: out_deg.at[pl.ds(r0, nr)], s)

    return pl.kernel(body, out_type=outs, mesh=mesh, scratch_types=scratch,
                     compiler_params=pltpu.CompilerParams(use_tc_tiling_on_sc=False))


_sc_agg_deg = _build_sc_agg(True)
_sc_agg = _build_sc_agg(False)


def _tc_pre_body(x_ref, a_ref, tout_ref):
    xb = x_ref[...]
    for q in range(4):
        tout_ref[q] = jnp.dot(xb, a_ref[q], preferred_element_type=jnp.float32)


def _tc_mid_body(agg_ref, deg_ref, h_ref, r_ref, b_ref, a_ref, hout_ref, tout_ref):
    agg = jnp.concatenate([agg_ref[0, 0], agg_ref[0, 1],
                           agg_ref[1, 0], agg_ref[1, 1]], axis=-1)
    deg = jnp.maximum(deg_ref[:, 0:1], 1.0)
    pre = (agg / deg
           + jnp.dot(h_ref[...], r_ref[...], preferred_element_type=jnp.float32)
           + b_ref[...])
    hn = jnp.where(pre > 0, pre, jnp.exp(pre) - 1.0)
    hout_ref[...] = hn
    for q in range(4):
        tout_ref[q] = jnp.dot(hn, a_ref[q], preferred_element_type=jnp.float32)


def _tc_final_body(agg_ref, deg_ref, h_ref, r_ref, b_ref, batch_ref,
                   fcw_ref, fcb_ref, out_ref, acc_ref):
    i = pl.program_id(0)

    @pl.when(i == 0)
    def _():
        acc_ref[...] = jnp.zeros((G, 128), jnp.float32)

    agg = jnp.concatenate([agg_ref[0, 0], agg_ref[0, 1],
                           agg_ref[1, 0], agg_ref[1, 1]], axis=-1)
    deg = jnp.maximum(deg_ref[:, 0:1], 1.0)
    pre = (agg / deg
           + jnp.dot(h_ref[...], r_ref[...], preferred_element_type=jnp.float32)
           + b_ref[...])
    h4 = jnp.where(pre > 0, pre, jnp.exp(pre) - 1.0)
    hext = jnp.concatenate([h4, jnp.ones((BLK, F), jnp.float32)], axis=1)
    onehot = (batch_ref[...] ==
              lax.broadcasted_iota(jnp.int32, (BLK, G), 1)).astype(jnp.float32)
    acc_ref[...] += lax.dot_general(onehot, hext, (((0,), (0,)), ((), ())),
                                    preferred_element_type=jnp.float32)

    @pl.when(i == pl.num_programs(0) - 1)
    def _():
        a = acc_ref[...]
        g = a[:, :F] / jnp.maximum(a[:, F:F + 1], 1.0)
        logits = jnp.dot(g, fcw_ref[...], preferred_element_type=jnp.float32) + fcb_ref[...]
        m = jnp.max(logits, axis=1, keepdims=True)
        lse = jnp.log(jnp.sum(jnp.exp(logits - m), axis=1, keepdims=True)) + m
        out_ref[...] = logits - lse


def _tc_pre(x, a):
    fin = x.shape[1]
    return pl.pallas_call(
        _tc_pre_body,
        grid=(NBLK,),
        in_specs=[
            pl.BlockSpec((BLK, fin), lambda i: (i, 0)),
            pl.BlockSpec((4, fin, HF), lambda i: (0, 0, 0)),
        ],
        out_specs=pl.BlockSpec((4, BLK, HF), lambda i: (0, i, 0)),
        out_shape=jax.ShapeDtypeStruct((4, N, HF), jnp.float32),
    )(x, a)


def _tc_mid(agg, deg, h, r, b, a):
    fin = h.shape[1]
    return pl.pallas_call(
        _tc_mid_body,
        grid=(NBLK,),
        in_specs=[
            pl.BlockSpec((2, 2, BLK, QF), lambda i: (0, 0, i, 0)),
            pl.BlockSpec((BLK, QF), lambda i: (i, 0)),
            pl.BlockSpec((BLK, fin), lambda i: (i, 0)),
            pl.BlockSpec((fin, F), lambda i: (0, 0)),
            pl.BlockSpec((1, F), lambda i: (0, 0)),
            pl.BlockSpec((4, F, HF), lambda i: (0, 0, 0)),
        ],
        out_specs=[
            pl.BlockSpec((BLK, F), lambda i: (i, 0)),
            pl.BlockSpec((4, BLK, HF), lambda i: (0, i, 0)),
        ],
        out_shape=[
            jax.ShapeDtypeStruct((N, F), jnp.float32),
            jax.ShapeDtypeStruct((4, N, HF), jnp.float32),
        ],
    )(agg, deg, h, r, b, a)


def _tc_final(agg, deg, h, r, b, batch2, fcw, fcb):
    return pl.pallas_call(
        _tc_final_body,
        grid=(NBLK,),
        in_specs=[
            pl.BlockSpec((2, 2, BLK, QF), lambda i: (0, 0, i, 0)),
            pl.BlockSpec((BLK, QF), lambda i: (i, 0)),
            pl.BlockSpec((BLK, F), lambda i: (i, 0)),
            pl.BlockSpec((F, F), lambda i: (0, 0)),
            pl.BlockSpec((1, F), lambda i: (0, 0)),
            pl.BlockSpec((BLK, 1), lambda i: (i, 0)),
            pl.BlockSpec((F, 6), lambda i: (0, 0)),
            pl.BlockSpec((1, 6), lambda i: (0, 0)),
        ],
        out_specs=pl.BlockSpec((G, 6), lambda i: (0, 0)),
        out_shape=jax.ShapeDtypeStruct((G, 6), jnp.float32),
        scratch_shapes=[pltpu.VMEM((G, 128), jnp.float32)],
    )(agg, deg, h, r, b, batch2, fcw, fcb)


def _amat(w):
    w0 = w[0]
    wd = w[1] - w[0]
    return jnp.stack([
        jnp.concatenate([w0[:, q * QF:(q + 1) * QF], wd[:, q * QF:(q + 1) * QF]],
                        axis=1)
        for q in range(4)
    ])


def kernel(x, edge_index, pseudo, batch, W1, R1, B1, W2, R2, B2, W3, R3, B3,
           W4, R4, B4, fcw, fcb):
    src = edge_index[0].astype(jnp.int32)
    dst = edge_index[1].astype(jnp.int32)
    u = pseudo[:, 0]
    npad = EPAD - E
    src = jnp.concatenate([src, jnp.zeros((npad,), jnp.int32)])
    dst = jnp.concatenate([dst, jnp.full((npad,), N, jnp.int32)])
    u = jnp.concatenate([u, jnp.zeros((npad,), jnp.float32)])
    src4 = jnp.concatenate([src, src + N, src + 2 * N, src + 3 * N])
    dst2 = dst.reshape(EPAD // CH, CH)
    batch2 = batch.astype(jnp.int32).reshape(N, 1)

    t = _tc_pre(x, _amat(W1))
    agg, deg16 = _sc_agg_deg(t.reshape(4 * N, 2 * QF), src4, dst2, u)
    agg, deg16 = agg[:, :, :N], deg16[:N]
    h, t = _tc_mid(agg, deg16, x, R1, B1.reshape(1, F), _amat(W2))
    agg = _sc_agg(t.reshape(4 * N, 2 * QF), src4, dst2, u)[:, :, :N]
    h, t = _tc_mid(agg, deg16, h, R2, B2.reshape(1, F), _amat(W3))
    agg = _sc_agg(t.reshape(4 * N, 2 * QF), src4, dst2, u)[:, :, :N]
    h, t = _tc_mid(agg, deg16, h, R3, B3.reshape(1, F), _amat(W4))
    agg = _sc_agg(t.reshape(4 * N, 2 * QF), src4, dst2, u)[:, :, :N]
    return _tc_final(agg, deg16, h, R4, B4.reshape(1, F), batch2, fcw,
                     fcb.reshape(1, 6))


# speedup vs baseline: 2.8487x; 1.4253x over previous
import functools

import jax
import jax.numpy as jnp
from jax import lax
from jax.experimental import pallas as pl
from jax.experimental.pallas import tpu as pltpu
from jax.experimental.pallas import tpu_sc as plsc

N = 50000
E = 800000
G = 16
F = 64
HF = 32
QF = 16
EPAD = 819200
NSC = 16
EW = EPAD // NSC
SUP = 1024
CH = 128
NSUB = SUP // CH
NCHUNK = EW // SUP
NACC = 50176
ZROWS = NACC // NSC
OROWS = N // NSC
BLK = 2000
NBLK = N // BLK


def _zero_z(z):
    def zrow(r, _):
        z[r, pl.ds(0, 16)] = jnp.zeros((16,), jnp.float32)
        return 0
    lax.fori_loop(0, SUP, zrow, 0)


def _zero_acc_slice(z, acc, s):
    zbase = s * ZROWS
    for j in range(3):
        pltpu.sync_copy(z, acc.at[pl.ds(zbase + j * SUP, SUP)])
    pltpu.sync_copy(z.at[pl.ds(0, ZROWS - 3 * SUP)],
                    acc.at[pl.ds(zbase + 3 * SUP, ZROWS - 3 * SUP)])


def _copy_out(acc, z, out_slice_fn, s):
    obase = s * ZROWS
    off = 0
    for nrows in (SUP, SUP, SUP, ZROWS - 3 * SUP):
        r0 = pl.multiple_of(obase + off, 8)
        pltpu.sync_copy(acc.at[pl.ds(r0, nrows)], z.at[pl.ds(0, nrows)])
        pltpu.sync_copy(z.at[pl.ds(0, nrows)], out_slice_fn(r0, nrows))
        off += nrows


def _build_sc_agg(want_deg):
    mesh = plsc.VectorSubcoreMesh(core_axis_name="c", subcore_axis_name="s")
    agg_t = jax.ShapeDtypeStruct((2, 2, NACC, QF), jnp.float32)
    outs = [agg_t, jax.ShapeDtypeStruct((NACC, QF), jnp.float32)] if want_deg else agg_t
    scratch = [
        pltpu.VMEM((SUP,), jnp.int32),
        pltpu.VMEM((SUP,), jnp.int32),
        pltpu.VMEM((NSUB, CH), jnp.int32),
        pltpu.VMEM((NSUB, CH), jnp.int32),
        pltpu.VMEM((SUP + 16,), jnp.float32),
        pltpu.VMEM((SUP + 16,), jnp.float32),
        pltpu.VMEM((SUP, 2 * QF), jnp.float32),
        pltpu.VMEM((SUP, 2 * QF), jnp.float32),
        pltpu.VMEM((SUP, QF), jnp.float32),
        pltpu.VMEM((SUP, QF), jnp.float32),
        pltpu.VMEM_SHARED((NACC, QF), jnp.float32),
        pltpu.SemaphoreType.DMA,
        pltpu.SemaphoreType.DMA,
        pltpu.SemaphoreType.DMA,
        pltpu.SemaphoreType.DMA,
    ]
    if want_deg:
        scratch.append(pltpu.VMEM((CH, QF), jnp.float32))

    def body(tflat, src4, dst2, u_all, out_agg, *rest):
        if want_deg:
            (out_deg, isrc0, isrc1, idst0, idst1, uv0, uv1, rows0, rows1,
             z0, z1, acc, gsem0, gsem1, ssem0, ssem1, ones) = rest
        else:
            (isrc0, isrc1, idst0, idst1, uv0, uv1, rows0, rows1,
             z0, z1, acc, gsem0, gsem1, ssem0, ssem1) = rest
        isrc = (isrc0, isrc1)
        idst = (idst0, idst1)
        uv = (uv0, uv1)
        rows = (rows0, rows1)
        z = (z0, z1)
        gsem = (gsem0, gsem1)
        ssem = (ssem0, ssem1)
        c = lax.axis_index("c")
        s = lax.axis_index("s")
        ebase = s * EW

        def agg_phase(p):
            _zero_z(z[0])
            _zero_acc_slice(z[0], acc, s)
            plsc.subcore_barrier()
            soff = (c + 2 * p) * EPAD

            def chunk(k, _):
                base = pl.multiple_of(ebase + k * SUP, SUP)
                pltpu.sync_copy(
                    src4.at[pl.ds(pl.multiple_of(soff + base, SUP), SUP)], isrc[0])
                pltpu.sync_copy(
                    dst2.at[pl.ds(pl.multiple_of(base // CH, 8), NSUB)], idst[0])
                pltpu.sync_copy(u_all.at[pl.ds(base, SUP)], uv[0].at[pl.ds(0, SUP)])
                cps = [
                    pltpu.async_copy(tflat.at[isrc[0].at[pl.ds(j * CH, CH)]],
                                     rows[0].at[pl.ds(j * CH, CH)], gsem[0])
                    for j in range(NSUB)
                ]
                cps2 = []
                for j in range(NSUB):
                    cps[j].wait()

                    @plsc.parallel_loop(0, CH // 16, unroll=2)
                    def group(g, j=j):
                        u16 = uv[0][pl.ds(j * CH + g * 16, 16)]
                        for i in range(16):
                            e = j * CH + g * 16 + i
                            ue = jnp.full((16,), u16[i], jnp.float32)
                            y0 = rows[0][e, pl.ds(0, 16)]
                            yd = rows[0][e, pl.ds(16, 16)]
                            z[0][e, pl.ds(0, 16)] = y0 + ue * yd
                    cps2.append(pltpu.async_copy(z[0].at[pl.ds(j * CH, CH)],
                                                 acc.at[idst[0].at[j]], ssem[0],
                                                 add=True))
                for cp in cps2:
                    cp.wait()
                return 0
            lax.fori_loop(0, NCHUNK, chunk, 0)
            plsc.subcore_barrier()
            _copy_out(acc, z[0], lambda r0, nr: out_agg.at[p, c, pl.ds(r0, nr)], s)

        agg_phase(0)
        agg_phase(1)

        if want_deg:
            @pl.when(c == 0)
            def _deg_phase():
                def orow(r, _):
                    ones[r, pl.ds(0, 16)] = jnp.ones((16,), jnp.float32)
                    return 0
                lax.fori_loop(0, CH, orow, 0)
                _zero_z(z[0])
                _zero_acc_slice(z[0], acc, s)
                plsc.subcore_barrier()

                def dchunk(k, _):
                    base = pl.multiple_of(ebase + k * SUP, SUP)
                    pltpu.sync_copy(
                        dst2.at[pl.ds(pl.multiple_of(base // CH, 8), NSUB)], idst[0])
                    dcps = [pltpu.async_copy(ones, acc.at[idst[0].at[j]], ssem[0],
                                             add=True)
                            for j in range(NSUB)]
                    for cp in dcps:
                        cp.wait()
                    return 0
                lax.fori_loop(0, NCHUNK, dchunk, 0)
                plsc.subcore_barrier()
                _copy_out(acc, z[0], lambda r0, nr: out_deg.at[pl.ds(r0, nr)], s)

    return pl.kernel(body, out_type=outs, mesh=mesh, scratch_types=scratch,
                     compiler_params=pltpu.CompilerParams(use_tc_tiling_on_sc=False))


_sc_agg_deg = _build_sc_agg(True)
_sc_agg = _build_sc_agg(False)


def _tc_pre_body(x_ref, a_ref, tout_ref):
    xb = x_ref[...]
    for q in range(4):
        tout_ref[q] = jnp.dot(xb, a_ref[q], preferred_element_type=jnp.float32)


def _tc_mid_body(agg_ref, deg_ref, h_ref, r_ref, b_ref, a_ref, hout_ref, tout_ref):
    agg = jnp.concatenate([agg_ref[0, 0], agg_ref[0, 1],
                           agg_ref[1, 0], agg_ref[1, 1]], axis=-1)
    deg = jnp.maximum(deg_ref[:, 0:1], 1.0)
    pre = (agg / deg
           + jnp.dot(h_ref[...], r_ref[...], preferred_element_type=jnp.float32)
           + b_ref[...])
    hn = jnp.where(pre > 0, pre, jnp.exp(pre) - 1.0)
    hout_ref[...] = hn
    for q in range(4):
        tout_ref[q] = jnp.dot(hn, a_ref[q], preferred_element_type=jnp.float32)


def _tc_final_body(agg_ref, deg_ref, h_ref, r_ref, b_ref, batch_ref,
                   fcw_ref, fcb_ref, out_ref, acc_ref):
    i = pl.program_id(0)

    @pl.when(i == 0)
    def _():
        acc_ref[...] = jnp.zeros((G, 128), jnp.float32)

    agg = jnp.concatenate([agg_ref[0, 0], agg_ref[0, 1],
                           agg_ref[1, 0], agg_ref[1, 1]], axis=-1)
    deg = jnp.maximum(deg_ref[:, 0:1], 1.0)
    pre = (agg / deg
           + jnp.dot(h_ref[...], r_ref[...], preferred_element_type=jnp.float32)
           + b_ref[...])
    h4 = jnp.where(pre > 0, pre, jnp.exp(pre) - 1.0)
    hext = jnp.concatenate([h4, jnp.ones((BLK, F), jnp.float32)], axis=1)
    onehot = (batch_ref[...] ==
              lax.broadcasted_iota(jnp.int32, (BLK, G), 1)).astype(jnp.float32)
    acc_ref[...] += lax.dot_general(onehot, hext, (((0,), (0,)), ((), ())),
                                    preferred_element_type=jnp.float32)

    @pl.when(i == pl.num_programs(0) - 1)
    def _():
        a = acc_ref[...]
        g = a[:, :F] / jnp.maximum(a[:, F:F + 1], 1.0)
        logits = jnp.dot(g, fcw_ref[...], preferred_element_type=jnp.float32) + fcb_ref[...]
        m = jnp.max(logits, axis=1, keepdims=True)
        lse = jnp.log(jnp.sum(jnp.exp(logits - m), axis=1, keepdims=True)) + m
        out_ref[...] = logits - lse


def _tc_pre(x, a):
    fin = x.shape[1]
    return pl.pallas_call(
        _tc_pre_body,
        grid=(NBLK,),
        in_specs=[
            pl.BlockSpec((BLK, fin), lambda i: (i, 0)),
            pl.BlockSpec((4, fin, HF), lambda i: (0, 0, 0)),
        ],
        out_specs=pl.BlockSpec((4, BLK, HF), lambda i: (0, i, 0)),
        out_shape=jax.ShapeDtypeStruct((4, N, HF), jnp.float32),
    )(x, a)


def _tc_mid(agg, deg, h, r, b, a):
    fin = h.shape[1]
    return pl.pallas_call(
        _tc_mid_body,
        grid=(NBLK,),
        in_specs=[
            pl.BlockSpec((2, 2, BLK, QF), lambda i: (0, 0, i, 0)),
            pl.BlockSpec((BLK, QF), lambda i: (i, 0)),
            pl.BlockSpec((BLK, fin), lambda i: (i, 0)),
            pl.BlockSpec((fin, F), lambda i: (0, 0)),
            pl.BlockSpec((1, F), lambda i: (0, 0)),
            pl.BlockSpec((4, F, HF), lambda i: (0, 0, 0)),
        ],
        out_specs=[
            pl.BlockSpec((BLK, F), lambda i: (i, 0)),
            pl.BlockSpec((4, BLK, HF), lambda i: (0, i, 0)),
        ],
        out_shape=[
            jax.ShapeDtypeStruct((N, F), jnp.float32),
            jax.ShapeDtypeStruct((4, N, HF), jnp.float32),
        ],
    )(agg, deg, h, r, b, a)


def _tc_final(agg, deg, h, r, b, batch2, fcw, fcb):
    return pl.pallas_call(
        _tc_final_body,
        grid=(NBLK,),
        in_specs=[
            pl.BlockSpec((2, 2, BLK, QF), lambda i: (0, 0, i, 0)),
            pl.BlockSpec((BLK, QF), lambda i: (i, 0)),
            pl.BlockSpec((BLK, F), lambda i: (i, 0)),
            pl.BlockSpec((F, F), lambda i: (0, 0)),
            pl.BlockSpec((1, F), lambda i: (0, 0)),
            pl.BlockSpec((BLK, 1), lambda i: (i, 0)),
            pl.BlockSpec((F, 6), lambda i: (0, 0)),
            pl.BlockSpec((1, 6), lambda i: (0, 0)),
        ],
        out_specs=pl.BlockSpec((G, 6), lambda i: (0, 0)),
        out_shape=jax.ShapeDtypeStruct((G, 6), jnp.float32),
        scratch_shapes=[pltpu.VMEM((G, 128), jnp.float32)],
    )(agg, deg, h, r, b, batch2, fcw, fcb)


def _amat(w):
    w0 = w[0]
    wd = w[1] - w[0]
    return jnp.stack([
        jnp.concatenate([w0[:, q * QF:(q + 1) * QF], wd[:, q * QF:(q + 1) * QF]],
                        axis=1)
        for q in range(4)
    ])


def kernel(x, edge_index, pseudo, batch, W1, R1, B1, W2, R2, B2, W3, R3, B3,
           W4, R4, B4, fcw, fcb):
    src = edge_index[0].astype(jnp.int32)
    dst = edge_index[1].astype(jnp.int32)
    u = pseudo[:, 0]
    npad = EPAD - E
    src = jnp.concatenate([src, jnp.zeros((npad,), jnp.int32)])
    dst = jnp.concatenate([dst, jnp.full((npad,), N, jnp.int32)])
    u = jnp.concatenate([u, jnp.zeros((npad,), jnp.float32)])
    src4 = jnp.concatenate([src, src + N, src + 2 * N, src + 3 * N])
    dst2 = dst.reshape(EPAD // CH, CH)
    batch2 = batch.astype(jnp.int32).reshape(N, 1)

    t = _tc_pre(x, _amat(W1))
    agg, deg16 = _sc_agg_deg(t.reshape(4 * N, 2 * QF), src4, dst2, u)
    agg, deg16 = agg[:, :, :N], deg16[:N]
    h, t = _tc_mid(agg, deg16, x, R1, B1.reshape(1, F), _amat(W2))
    agg = _sc_agg(t.reshape(4 * N, 2 * QF), src4, dst2, u)[:, :, :N]
    h, t = _tc_mid(agg, deg16, h, R2, B2.reshape(1, F), _amat(W3))
    agg = _sc_agg(t.reshape(4 * N, 2 * QF), src4, dst2, u)[:, :, :N]
    h, t = _tc_mid(agg, deg16, h, R3, B3.reshape(1, F), _amat(W4))
    agg = _sc_agg(t.reshape(4 * N, 2 * QF), src4, dst2, u)[:, :, :N]
    return _tc_final(agg, deg16, h, R4, B4.reshape(1, F), batch2, fcw,
                     fcb.reshape(1, 6))


# speedup vs baseline: 3.0274x; 1.0627x over previous
import functools

import jax
import jax.numpy as jnp
from jax import lax
from jax.experimental import pallas as pl
from jax.experimental.pallas import tpu as pltpu
from jax.experimental.pallas import tpu_sc as plsc

N = 50000
E = 800000
G = 16
F = 64
HF = 32
QF = 16
EPAD = 819200
NSC = 16
EW = EPAD // NSC
SUP = 1024
CH = 128
NSUB = SUP // CH
NCHUNK = EW // SUP
NACC = 50176
ZROWS = NACC // NSC
OROWS = N // NSC
BLK = 2000
NBLK = N // BLK


def _zero_z(z):
    def zrow(r, _):
        z[r, pl.ds(0, 16)] = jnp.zeros((16,), jnp.float32)
        return 0
    lax.fori_loop(0, SUP, zrow, 0)


def _zero_acc_slice(z, acc, s):
    zbase = s * ZROWS
    for j in range(3):
        pltpu.sync_copy(z, acc.at[pl.ds(zbase + j * SUP, SUP)])
    pltpu.sync_copy(z.at[pl.ds(0, ZROWS - 3 * SUP)],
                    acc.at[pl.ds(zbase + 3 * SUP, ZROWS - 3 * SUP)])


def _copy_out(acc, z, out_slice_fn, s):
    obase = s * ZROWS
    off = 0
    for nrows in (SUP, SUP, SUP, ZROWS - 3 * SUP):
        r0 = pl.multiple_of(obase + off, 8)
        pltpu.sync_copy(acc.at[pl.ds(r0, nrows)], z.at[pl.ds(0, nrows)])
        pltpu.sync_copy(z.at[pl.ds(0, nrows)], out_slice_fn(r0, nrows))
        off += nrows


def _build_sc_agg(want_deg):
    mesh = plsc.VectorSubcoreMesh(core_axis_name="c", subcore_axis_name="s")
    agg_t = jax.ShapeDtypeStruct((2, 2, NACC, QF), jnp.float32)
    outs = [agg_t, jax.ShapeDtypeStruct((NACC, QF), jnp.float32)] if want_deg else agg_t
    scratch = [
        pltpu.VMEM((SUP,), jnp.int32),
        pltpu.VMEM((SUP,), jnp.int32),
        pltpu.VMEM((NSUB, CH), jnp.int32),
        pltpu.VMEM((NSUB, CH), jnp.int32),
        pltpu.VMEM((SUP + 16,), jnp.float32),
        pltpu.VMEM((SUP + 16,), jnp.float32),
        pltpu.VMEM((SUP, 2 * QF), jnp.float32),
        pltpu.VMEM((SUP, 2 * QF), jnp.float32),
        pltpu.VMEM((SUP, QF), jnp.float32),
        pltpu.VMEM((SUP, QF), jnp.float32),
        pltpu.VMEM_SHARED((NACC, QF), jnp.float32),
        pltpu.SemaphoreType.DMA,
        pltpu.SemaphoreType.DMA,
        pltpu.SemaphoreType.DMA,
        pltpu.SemaphoreType.DMA,
    ]
    if want_deg:
        scratch.append(pltpu.VMEM((CH, QF), jnp.float32))

    def body(tflat, src4, dst2, u_all, out_agg, *rest):
        if want_deg:
            (out_deg, isrc0, isrc1, idst0, idst1, uv0, uv1, rows0, rows1,
             z0, z1, acc, gsem0, gsem1, ssem0, ssem1, ones) = rest
        else:
            (isrc0, isrc1, idst0, idst1, uv0, uv1, rows0, rows1,
             z0, z1, acc, gsem0, gsem1, ssem0, ssem1) = rest
        isrc = (isrc0, isrc1)
        idst = (idst0, idst1)
        uv = (uv0, uv1)
        rows = (rows0, rows1)
        z = (z0, z1)
        gsem = (gsem0, gsem1)
        ssem = (ssem0, ssem1)
        c = lax.axis_index("c")
        s = lax.axis_index("s")
        ebase = s * EW

        def agg_phase(p):
            _zero_z(z[0])
            _zero_acc_slice(z[0], acc, s)
            plsc.subcore_barrier()
            soff = (c + 2 * p) * EPAD

            def chunk(k, _):
                base = pl.multiple_of(ebase + k * SUP, SUP)
                pltpu.sync_copy(
                    src4.at[pl.ds(pl.multiple_of(soff + base, SUP), SUP)], isrc[0])
                pltpu.sync_copy(
                    dst2.at[pl.ds(pl.multiple_of(base // CH, 8), NSUB)], idst[0])
                pltpu.sync_copy(u_all.at[pl.ds(base, SUP)], uv[0].at[pl.ds(0, SUP)])
                cps = [
                    pltpu.async_copy(tflat.at[isrc[0].at[pl.ds(j * CH, CH)]],
                                     rows[0].at[pl.ds(j * CH, CH)], gsem[0])
                    for j in range(NSUB)
                ]
                cps2 = []
                for j in range(NSUB):
                    cps[j].wait()

                    @plsc.parallel_loop(0, CH // 16, unroll=4)
                    def group(g, j=j):
                        u16 = uv[0][pl.ds(j * CH + g * 16, 16)]
                        for i in range(16):
                            e = j * CH + g * 16 + i
                            ue = jnp.full((16,), u16[i], jnp.float32)
                            y0 = rows[0][e, pl.ds(0, 16)]
                            yd = rows[0][e, pl.ds(16, 16)]
                            z[0][e, pl.ds(0, 16)] = y0 + ue * yd
                    cps2.append(pltpu.async_copy(z[0].at[pl.ds(j * CH, CH)],
                                                 acc.at[idst[0].at[j]], ssem[0],
                                                 add=True))
                for cp in cps2:
                    cp.wait()
                return 0
            lax.fori_loop(0, NCHUNK, chunk, 0)
            plsc.subcore_barrier()
            _copy_out(acc, z[0], lambda r0, nr: out_agg.at[p, c, pl.ds(r0, nr)], s)

        agg_phase(0)
        agg_phase(1)

        if want_deg:
            @pl.when(c == 0)
            def _deg_phase():
                def orow(r, _):
                    ones[r, pl.ds(0, 16)] = jnp.ones((16,), jnp.float32)
                    return 0
                lax.fori_loop(0, CH, orow, 0)
                _zero_z(z[0])
                _zero_acc_slice(z[0], acc, s)
                plsc.subcore_barrier()

                def dchunk(k, _):
                    base = pl.multiple_of(ebase + k * SUP, SUP)
                    pltpu.sync_copy(
                        dst2.at[pl.ds(pl.multiple_of(base // CH, 8), NSUB)], idst[0])
                    dcps = [pltpu.async_copy(ones, acc.at[idst[0].at[j]], ssem[0],
                                             add=True)
                            for j in range(NSUB)]
                    for cp in dcps:
                        cp.wait()
                    return 0
                lax.fori_loop(0, NCHUNK, dchunk, 0)
                plsc.subcore_barrier()
                _copy_out(acc, z[0], lambda r0, nr: out_deg.at[pl.ds(r0, nr)], s)

    return pl.kernel(body, out_type=outs, mesh=mesh, scratch_types=scratch,
                     compiler_params=pltpu.CompilerParams(use_tc_tiling_on_sc=False))


_sc_agg_deg = _build_sc_agg(True)
_sc_agg = _build_sc_agg(False)


def _tc_pre_body(x_ref, a_ref, tout_ref):
    xb = x_ref[...]
    for q in range(4):
        tout_ref[q] = jnp.dot(xb, a_ref[q], preferred_element_type=jnp.float32)


def _tc_mid_body(agg_ref, deg_ref, h_ref, r_ref, b_ref, a_ref, hout_ref, tout_ref):
    agg = jnp.concatenate([agg_ref[0, 0], agg_ref[0, 1],
                           agg_ref[1, 0], agg_ref[1, 1]], axis=-1)
    deg = jnp.maximum(deg_ref[:, 0:1], 1.0)
    pre = (agg / deg
           + jnp.dot(h_ref[...], r_ref[...], preferred_element_type=jnp.float32)
           + b_ref[...])
    hn = jnp.where(pre > 0, pre, jnp.exp(pre) - 1.0)
    hout_ref[...] = hn
    for q in range(4):
        tout_ref[q] = jnp.dot(hn, a_ref[q], preferred_element_type=jnp.float32)


def _tc_final_body(agg_ref, deg_ref, h_ref, r_ref, b_ref, batch_ref,
                   fcw_ref, fcb_ref, out_ref, acc_ref):
    i = pl.program_id(0)

    @pl.when(i == 0)
    def _():
        acc_ref[...] = jnp.zeros((G, 128), jnp.float32)

    agg = jnp.concatenate([agg_ref[0, 0], agg_ref[0, 1],
                           agg_ref[1, 0], agg_ref[1, 1]], axis=-1)
    deg = jnp.maximum(deg_ref[:, 0:1], 1.0)
    pre = (agg / deg
           + jnp.dot(h_ref[...], r_ref[...], preferred_element_type=jnp.float32)
           + b_ref[...])
    h4 = jnp.where(pre > 0, pre, jnp.exp(pre) - 1.0)
    hext = jnp.concatenate([h4, jnp.ones((BLK, F), jnp.float32)], axis=1)
    onehot = (batch_ref[...] ==
              lax.broadcasted_iota(jnp.int32, (BLK, G), 1)).astype(jnp.float32)
    acc_ref[...] += lax.dot_general(onehot, hext, (((0,), (0,)), ((), ())),
                                    preferred_element_type=jnp.float32)

    @pl.when(i == pl.num_programs(0) - 1)
    def _():
        a = acc_ref[...]
        g = a[:, :F] / jnp.maximum(a[:, F:F + 1], 1.0)
        logits = jnp.dot(g, fcw_ref[...], preferred_element_type=jnp.float32) + fcb_ref[...]
        m = jnp.max(logits, axis=1, keepdims=True)
        lse = jnp.log(jnp.sum(jnp.exp(logits - m), axis=1, keepdims=True)) + m
        out_ref[...] = logits - lse


def _tc_pre(x, a):
    fin = x.shape[1]
    return pl.pallas_call(
        _tc_pre_body,
        grid=(NBLK,),
        in_specs=[
            pl.BlockSpec((BLK, fin), lambda i: (i, 0)),
            pl.BlockSpec((4, fin, HF), lambda i: (0, 0, 0)),
        ],
        out_specs=pl.BlockSpec((4, BLK, HF), lambda i: (0, i, 0)),
        out_shape=jax.ShapeDtypeStruct((4, N, HF), jnp.float32),
    )(x, a)


def _tc_mid(agg, deg, h, r, b, a):
    fin = h.shape[1]
    return pl.pallas_call(
        _tc_mid_body,
        grid=(NBLK,),
        in_specs=[
            pl.BlockSpec((2, 2, BLK, QF), lambda i: (0, 0, i, 0)),
            pl.BlockSpec((BLK, QF), lambda i: (i, 0)),
            pl.BlockSpec((BLK, fin), lambda i: (i, 0)),
            pl.BlockSpec((fin, F), lambda i: (0, 0)),
            pl.BlockSpec((1, F), lambda i: (0, 0)),
            pl.BlockSpec((4, F, HF), lambda i: (0, 0, 0)),
        ],
        out_specs=[
            pl.BlockSpec((BLK, F), lambda i: (i, 0)),
            pl.BlockSpec((4, BLK, HF), lambda i: (0, i, 0)),
        ],
        out_shape=[
            jax.ShapeDtypeStruct((N, F), jnp.float32),
            jax.ShapeDtypeStruct((4, N, HF), jnp.float32),
        ],
    )(agg, deg, h, r, b, a)


def _tc_final(agg, deg, h, r, b, batch2, fcw, fcb):
    return pl.pallas_call(
        _tc_final_body,
        grid=(NBLK,),
        in_specs=[
            pl.BlockSpec((2, 2, BLK, QF), lambda i: (0, 0, i, 0)),
            pl.BlockSpec((BLK, QF), lambda i: (i, 0)),
            pl.BlockSpec((BLK, F), lambda i: (i, 0)),
            pl.BlockSpec((F, F), lambda i: (0, 0)),
            pl.BlockSpec((1, F), lambda i: (0, 0)),
            pl.BlockSpec((BLK, 1), lambda i: (i, 0)),
            pl.BlockSpec((F, 6), lambda i: (0, 0)),
            pl.BlockSpec((1, 6), lambda i: (0, 0)),
        ],
        out_specs=pl.BlockSpec((G, 6), lambda i: (0, 0)),
        out_shape=jax.ShapeDtypeStruct((G, 6), jnp.float32),
        scratch_shapes=[pltpu.VMEM((G, 128), jnp.float32)],
    )(agg, deg, h, r, b, batch2, fcw, fcb)


def _amat(w):
    w0 = w[0]
    wd = w[1] - w[0]
    return jnp.stack([
        jnp.concatenate([w0[:, q * QF:(q + 1) * QF], wd[:, q * QF:(q + 1) * QF]],
                        axis=1)
        for q in range(4)
    ])


def kernel(x, edge_index, pseudo, batch, W1, R1, B1, W2, R2, B2, W3, R3, B3,
           W4, R4, B4, fcw, fcb):
    src = edge_index[0].astype(jnp.int32)
    dst = edge_index[1].astype(jnp.int32)
    u = pseudo[:, 0]
    npad = EPAD - E
    src = jnp.concatenate([src, jnp.zeros((npad,), jnp.int32)])
    dst = jnp.concatenate([dst, jnp.full((npad,), N, jnp.int32)])
    u = jnp.concatenate([u, jnp.zeros((npad,), jnp.float32)])
    src4 = jnp.concatenate([src, src + N, src + 2 * N, src + 3 * N])
    dst2 = dst.reshape(EPAD // CH, CH)
    batch2 = batch.astype(jnp.int32).reshape(N, 1)

    t = _tc_pre(x, _amat(W1))
    agg, deg16 = _sc_agg_deg(t.reshape(4 * N, 2 * QF), src4, dst2, u)
    agg, deg16 = agg[:, :, :N], deg16[:N]
    h, t = _tc_mid(agg, deg16, x, R1, B1.reshape(1, F), _amat(W2))
    agg = _sc_agg(t.reshape(4 * N, 2 * QF), src4, dst2, u)[:, :, :N]
    h, t = _tc_mid(agg, deg16, h, R2, B2.reshape(1, F), _amat(W3))
    agg = _sc_agg(t.reshape(4 * N, 2 * QF), src4, dst2, u)[:, :, :N]
    h, t = _tc_mid(agg, deg16, h, R3, B3.reshape(1, F), _amat(W4))
    agg = _sc_agg(t.reshape(4 * N, 2 * QF), src4, dst2, u)[:, :, :N]
    return _tc_final(agg, deg16, h, R4, B4.reshape(1, F), batch2, fcw,
                     fcb.reshape(1, 6))


# speedup vs baseline: 3.1995x; 1.0569x over previous
import functools

import jax
import jax.numpy as jnp
from jax import lax
from jax.experimental import pallas as pl
from jax.experimental.pallas import tpu as pltpu
from jax.experimental.pallas import tpu_sc as plsc

N = 50000
E = 800000
G = 16
F = 64
HF = 32
QF = 16
EPAD = 819200
NSC = 16
EW = EPAD // NSC
SUP = 1024
CH = 128
NSUB = SUP // CH
NCHUNK = EW // SUP
NACC = 50176
ZROWS = NACC // NSC
OROWS = N // NSC
BLK = 2000
NBLK = N // BLK


def _zero_z(z):
    def zrow(r, _):
        z[r, pl.ds(0, 16)] = jnp.zeros((16,), jnp.float32)
        return 0
    lax.fori_loop(0, SUP, zrow, 0)


def _zero_acc_slice(z, acc, s):
    zbase = s * ZROWS
    for j in range(3):
        pltpu.sync_copy(z, acc.at[pl.ds(zbase + j * SUP, SUP)])
    pltpu.sync_copy(z.at[pl.ds(0, ZROWS - 3 * SUP)],
                    acc.at[pl.ds(zbase + 3 * SUP, ZROWS - 3 * SUP)])


def _copy_out(acc, z, out_slice_fn, s):
    obase = s * ZROWS
    off = 0
    for nrows in (SUP, SUP, SUP, ZROWS - 3 * SUP):
        r0 = pl.multiple_of(obase + off, 8)
        pltpu.sync_copy(acc.at[pl.ds(r0, nrows)], z.at[pl.ds(0, nrows)])
        pltpu.sync_copy(z.at[pl.ds(0, nrows)], out_slice_fn(r0, nrows))
        off += nrows


def _build_sc_agg(want_deg):
    mesh = plsc.VectorSubcoreMesh(core_axis_name="c", subcore_axis_name="s")
    agg_t = jax.ShapeDtypeStruct((2, 2, NACC, QF), jnp.float32)
    outs = [agg_t, jax.ShapeDtypeStruct((NACC, QF), jnp.float32)] if want_deg else agg_t
    scratch = [
        pltpu.VMEM((3 * NSUB, CH), jnp.int32),
        pltpu.VMEM((SUP, 2 * QF), jnp.float32),
        pltpu.VMEM((SUP, QF), jnp.float32),
        pltpu.VMEM_SHARED((NACC, QF), jnp.float32),
        pltpu.SemaphoreType.DMA,
        pltpu.SemaphoreType.DMA,
    ]
    if want_deg:
        scratch.append(pltpu.VMEM((CH, QF), jnp.float32))

    def body(tflat, comb_all, out_agg, *rest):
        if want_deg:
            out_deg, comb, rows0, z0, acc, gsem0, ssem0, ones = rest
        else:
            comb, rows0, z0, acc, gsem0, ssem0 = rest
        rows = (rows0,)
        z = (z0,)
        gsem = (gsem0,)
        ssem = (ssem0,)
        c = lax.axis_index("c")
        s = lax.axis_index("s")
        ebase = s * EW

        def agg_phase(p):
            _zero_z(z[0])
            _zero_acc_slice(z[0], acc, s)
            plsc.subcore_barrier()
            q = c + 2 * p

            def chunk(k, _):
                scidx = s * NCHUNK + k
                pltpu.sync_copy(comb_all.at[q, scidx], comb)
                cps = [
                    pltpu.async_copy(tflat.at[comb.at[j]],
                                     rows[0].at[pl.ds(j * CH, CH)], gsem[0])
                    for j in range(NSUB)
                ]
                cps2 = []
                for j in range(NSUB):
                    cps[j].wait()

                    @plsc.parallel_loop(0, CH // 16, unroll=4)
                    def group(g, j=j):
                        u16 = jax.lax.bitcast_convert_type(
                            comb[2 * NSUB + j, pl.ds(g * 16, 16)], jnp.float32)
                        for i in range(16):
                            e = j * CH + g * 16 + i
                            ue = jnp.full((16,), u16[i], jnp.float32)
                            y0 = rows[0][e, pl.ds(0, 16)]
                            yd = rows[0][e, pl.ds(16, 16)]
                            z[0][e, pl.ds(0, 16)] = y0 + ue * yd
                    cps2.append(pltpu.async_copy(z[0].at[pl.ds(j * CH, CH)],
                                                 acc.at[comb.at[NSUB + j]], ssem[0],
                                                 add=True))
                for cp in cps2:
                    cp.wait()
                return 0
            lax.fori_loop(0, NCHUNK, chunk, 0)
            plsc.subcore_barrier()
            _copy_out(acc, z[0], lambda r0, nr: out_agg.at[p, c, pl.ds(r0, nr)], s)

        agg_phase(0)
        agg_phase(1)

        if want_deg:
            @pl.when(c == 0)
            def _deg_phase():
                def orow(r, _):
                    ones[r, pl.ds(0, 16)] = jnp.ones((16,), jnp.float32)
                    return 0
                lax.fori_loop(0, CH, orow, 0)
                _zero_z(z[0])
                _zero_acc_slice(z[0], acc, s)
                plsc.subcore_barrier()

                def dchunk(k, _):
                    scidx = s * NCHUNK + k
                    pltpu.sync_copy(comb_all.at[0, scidx], comb)
                    dcps = [pltpu.async_copy(ones, acc.at[comb.at[NSUB + j]], ssem[0],
                                             add=True)
                            for j in range(NSUB)]
                    for cp in dcps:
                        cp.wait()
                    return 0
                lax.fori_loop(0, NCHUNK, dchunk, 0)
                plsc.subcore_barrier()
                _copy_out(acc, z[0], lambda r0, nr: out_deg.at[pl.ds(r0, nr)], s)

    return pl.kernel(body, out_type=outs, mesh=mesh, scratch_types=scratch,
                     compiler_params=pltpu.CompilerParams(use_tc_tiling_on_sc=False))


_sc_agg_deg = _build_sc_agg(True)
_sc_agg = _build_sc_agg(False)


def _tc_pre_body(x_ref, a_ref, tout_ref):
    xb = x_ref[...]
    for q in range(4):
        tout_ref[q] = jnp.dot(xb, a_ref[q], preferred_element_type=jnp.float32)


def _tc_mid_body(agg_ref, deg_ref, h_ref, r_ref, b_ref, a_ref, hout_ref, tout_ref):
    agg = jnp.concatenate([agg_ref[0, 0], agg_ref[0, 1],
                           agg_ref[1, 0], agg_ref[1, 1]], axis=-1)
    deg = jnp.maximum(deg_ref[:, 0:1], 1.0)
    pre = (agg / deg
           + jnp.dot(h_ref[...], r_ref[...], preferred_element_type=jnp.float32)
           + b_ref[...])
    hn = jnp.where(pre > 0, pre, jnp.exp(pre) - 1.0)
    hout_ref[...] = hn
    for q in range(4):
        tout_ref[q] = jnp.dot(hn, a_ref[q], preferred_element_type=jnp.float32)


def _tc_final_body(agg_ref, deg_ref, h_ref, r_ref, b_ref, batch_ref,
                   fcw_ref, fcb_ref, out_ref, acc_ref):
    i = pl.program_id(0)

    @pl.when(i == 0)
    def _():
        acc_ref[...] = jnp.zeros((G, 128), jnp.float32)

    agg = jnp.concatenate([agg_ref[0, 0], agg_ref[0, 1],
                           agg_ref[1, 0], agg_ref[1, 1]], axis=-1)
    deg = jnp.maximum(deg_ref[:, 0:1], 1.0)
    pre = (agg / deg
           + jnp.dot(h_ref[...], r_ref[...], preferred_element_type=jnp.float32)
           + b_ref[...])
    h4 = jnp.where(pre > 0, pre, jnp.exp(pre) - 1.0)
    hext = jnp.concatenate([h4, jnp.ones((BLK, F), jnp.float32)], axis=1)
    onehot = (batch_ref[...] ==
              lax.broadcasted_iota(jnp.int32, (BLK, G), 1)).astype(jnp.float32)
    acc_ref[...] += lax.dot_general(onehot, hext, (((0,), (0,)), ((), ())),
                                    preferred_element_type=jnp.float32)

    @pl.when(i == pl.num_programs(0) - 1)
    def _():
        a = acc_ref[...]
        g = a[:, :F] / jnp.maximum(a[:, F:F + 1], 1.0)
        logits = jnp.dot(g, fcw_ref[...], preferred_element_type=jnp.float32) + fcb_ref[...]
        m = jnp.max(logits, axis=1, keepdims=True)
        lse = jnp.log(jnp.sum(jnp.exp(logits - m), axis=1, keepdims=True)) + m
        out_ref[...] = logits - lse


def _tc_pre(x, a):
    fin = x.shape[1]
    return pl.pallas_call(
        _tc_pre_body,
        grid=(NBLK,),
        in_specs=[
            pl.BlockSpec((BLK, fin), lambda i: (i, 0)),
            pl.BlockSpec((4, fin, HF), lambda i: (0, 0, 0)),
        ],
        out_specs=pl.BlockSpec((4, BLK, HF), lambda i: (0, i, 0)),
        out_shape=jax.ShapeDtypeStruct((4, N, HF), jnp.float32),
    )(x, a)


def _tc_mid(agg, deg, h, r, b, a):
    fin = h.shape[1]
    return pl.pallas_call(
        _tc_mid_body,
        grid=(NBLK,),
        in_specs=[
            pl.BlockSpec((2, 2, BLK, QF), lambda i: (0, 0, i, 0)),
            pl.BlockSpec((BLK, QF), lambda i: (i, 0)),
            pl.BlockSpec((BLK, fin), lambda i: (i, 0)),
            pl.BlockSpec((fin, F), lambda i: (0, 0)),
            pl.BlockSpec((1, F), lambda i: (0, 0)),
            pl.BlockSpec((4, F, HF), lambda i: (0, 0, 0)),
        ],
        out_specs=[
            pl.BlockSpec((BLK, F), lambda i: (i, 0)),
            pl.BlockSpec((4, BLK, HF), lambda i: (0, i, 0)),
        ],
        out_shape=[
            jax.ShapeDtypeStruct((N, F), jnp.float32),
            jax.ShapeDtypeStruct((4, N, HF), jnp.float32),
        ],
    )(agg, deg, h, r, b, a)


def _tc_final(agg, deg, h, r, b, batch2, fcw, fcb):
    return pl.pallas_call(
        _tc_final_body,
        grid=(NBLK,),
        in_specs=[
            pl.BlockSpec((2, 2, BLK, QF), lambda i: (0, 0, i, 0)),
            pl.BlockSpec((BLK, QF), lambda i: (i, 0)),
            pl.BlockSpec((BLK, F), lambda i: (i, 0)),
            pl.BlockSpec((F, F), lambda i: (0, 0)),
            pl.BlockSpec((1, F), lambda i: (0, 0)),
            pl.BlockSpec((BLK, 1), lambda i: (i, 0)),
            pl.BlockSpec((F, 6), lambda i: (0, 0)),
            pl.BlockSpec((1, 6), lambda i: (0, 0)),
        ],
        out_specs=pl.BlockSpec((G, 6), lambda i: (0, 0)),
        out_shape=jax.ShapeDtypeStruct((G, 6), jnp.float32),
        scratch_shapes=[pltpu.VMEM((G, 128), jnp.float32)],
    )(agg, deg, h, r, b, batch2, fcw, fcb)


def _amat(w):
    w0 = w[0]
    wd = w[1] - w[0]
    return jnp.stack([
        jnp.concatenate([w0[:, q * QF:(q + 1) * QF], wd[:, q * QF:(q + 1) * QF]],
                        axis=1)
        for q in range(4)
    ])


def kernel(x, edge_index, pseudo, batch, W1, R1, B1, W2, R2, B2, W3, R3, B3,
           W4, R4, B4, fcw, fcb):
    src = edge_index[0].astype(jnp.int32)
    dst = edge_index[1].astype(jnp.int32)
    u = pseudo[:, 0]
    npad = EPAD - E
    src = jnp.concatenate([src, jnp.zeros((npad,), jnp.int32)])
    dst = jnp.concatenate([dst, jnp.full((npad,), N, jnp.int32)])
    u = jnp.concatenate([u, jnp.zeros((npad,), jnp.float32)])
    nsc = EPAD // SUP
    dstm = dst.reshape(nsc, NSUB, CH)
    um = jax.lax.bitcast_convert_type(u, jnp.int32).reshape(nsc, NSUB, CH)
    comb_all = jnp.stack([
        jnp.concatenate([(src + qq * N).reshape(nsc, NSUB, CH), dstm, um], axis=1)
        for qq in range(4)
    ])
    batch2 = batch.astype(jnp.int32).reshape(N, 1)

    t = _tc_pre(x, _amat(W1))
    agg, deg16 = _sc_agg_deg(t.reshape(4 * N, 2 * QF), comb_all)
    agg, deg16 = agg[:, :, :N], deg16[:N]
    h, t = _tc_mid(agg, deg16, x, R1, B1.reshape(1, F), _amat(W2))
    agg = _sc_agg(t.reshape(4 * N, 2 * QF), comb_all)[:, :, :N]
    h, t = _tc_mid(agg, deg16, h, R2, B2.reshape(1, F), _amat(W3))
    agg = _sc_agg(t.reshape(4 * N, 2 * QF), comb_all)[:, :, :N]
    h, t = _tc_mid(agg, deg16, h, R3, B3.reshape(1, F), _amat(W4))
    agg = _sc_agg(t.reshape(4 * N, 2 * QF), comb_all)[:, :, :N]
    return _tc_final(agg, deg16, h, R4, B4.reshape(1, F), batch2, fcw,
                     fcb.reshape(1, 6))


# speedup vs baseline: 3.4293x; 1.0718x over previous
import functools

import jax
import jax.numpy as jnp
from jax import lax
from jax.experimental import pallas as pl
from jax.experimental.pallas import tpu as pltpu
from jax.experimental.pallas import tpu_sc as plsc

N = 50000
E = 800000
G = 16
F = 64
HF = 32
QF = 16
EPAD = 819200
NSC = 16
EW = EPAD // NSC
SUP = 1024
CH = 128
NSUB = SUP // CH
NCHUNK = EW // SUP
NACC = 50176
ZROWS = NACC // NSC
OROWS = N // NSC
BLK = 2000
NBLK = N // BLK


def _zero_z(z):
    def zrow(r, _):
        z[r, pl.ds(0, 16)] = jnp.zeros((16,), jnp.float32)
        return 0
    lax.fori_loop(0, SUP, zrow, 0)


def _zero_acc_slice(z, acc, s):
    zbase = s * ZROWS
    for j in range(3):
        pltpu.sync_copy(z, acc.at[pl.ds(zbase + j * SUP, SUP)])
    pltpu.sync_copy(z.at[pl.ds(0, ZROWS - 3 * SUP)],
                    acc.at[pl.ds(zbase + 3 * SUP, ZROWS - 3 * SUP)])


def _copy_out(acc, z, out_slice_fn, s):
    obase = s * ZROWS
    off = 0
    for nrows in (SUP, SUP, SUP, ZROWS - 3 * SUP):
        r0 = pl.multiple_of(obase + off, 8)
        pltpu.sync_copy(acc.at[pl.ds(r0, nrows)], z.at[pl.ds(0, nrows)])
        pltpu.sync_copy(z.at[pl.ds(0, nrows)], out_slice_fn(r0, nrows))
        off += nrows


def _build_sc_agg(want_deg):
    mesh = plsc.VectorSubcoreMesh(core_axis_name="c", subcore_axis_name="s")
    agg_t = jax.ShapeDtypeStruct((2, 2, NACC, QF), jnp.float32)
    outs = [agg_t, jax.ShapeDtypeStruct((NACC, QF), jnp.float32)] if want_deg else agg_t
    scratch = [
        pltpu.VMEM((3 * NSUB, CH), jnp.int32),
        pltpu.VMEM((3 * NSUB, CH), jnp.int32),
        pltpu.VMEM((SUP, 2 * QF), jnp.float32),
        pltpu.VMEM((SUP, QF), jnp.float32),
        pltpu.VMEM_SHARED((NACC, QF), jnp.float32),
        pltpu.SemaphoreType.DMA,
        pltpu.SemaphoreType.DMA,
        pltpu.SemaphoreType.DMA,
    ]
    if want_deg:
        scratch.append(pltpu.VMEM((CH, QF), jnp.float32))

    def body(tflat, comb_all, out_agg, *rest):
        if want_deg:
            out_deg, comb0, comb1, rows0, z0, acc, gsem0, ssem0, csem, ones = rest
        else:
            comb0, comb1, rows0, z0, acc, gsem0, ssem0, csem = rest
        combs = (comb0, comb1)
        rows = (rows0,)
        z = (z0,)
        gsem = (gsem0,)
        ssem = (ssem0,)
        c = lax.axis_index("c")
        s = lax.axis_index("s")
        ebase = s * EW

        def agg_phase(p):
            _zero_z(z[0])
            _zero_acc_slice(z[0], acc, s)
            plsc.subcore_barrier()
            q = c + 2 * p

            def run_chunk(k, comb, nxt_cp):
                cps = [
                    pltpu.async_copy(tflat.at[comb.at[j]],
                                     rows[0].at[pl.ds(j * CH, CH)], gsem[0])
                    for j in range(NSUB)
                ]
                cps2 = []
                for j in range(NSUB):
                    cps[j].wait()

                    @plsc.parallel_loop(0, CH // 16, unroll=4)
                    def group(g, j=j):
                        u16 = jax.lax.bitcast_convert_type(
                            comb[2 * NSUB + j, pl.ds(g * 16, 16)], jnp.float32)
                        for i in range(16):
                            e = j * CH + g * 16 + i
                            ue = jnp.full((16,), u16[i], jnp.float32)
                            y0 = rows[0][e, pl.ds(0, 16)]
                            yd = rows[0][e, pl.ds(16, 16)]
                            z[0][e, pl.ds(0, 16)] = y0 + ue * yd
                    cps2.append(pltpu.async_copy(z[0].at[pl.ds(j * CH, CH)],
                                                 acc.at[comb.at[NSUB + j]], ssem[0],
                                                 add=True))
                for cp in cps2:
                    cp.wait()
                nxt_cp.wait()

            pltpu.sync_copy(comb_all.at[q, s * NCHUNK], combs[0])

            def pair(kk, _):
                for h in range(2):
                    k = 2 * kk + h
                    nxt_cp = pltpu.async_copy(
                        comb_all.at[q, s * NCHUNK + k + 1], combs[1 - h], csem)
                    run_chunk(k, combs[h], nxt_cp)
                return 0
            lax.fori_loop(0, NCHUNK // 2, pair, 0)
            plsc.subcore_barrier()
            _copy_out(acc, z[0], lambda r0, nr: out_agg.at[p, c, pl.ds(r0, nr)], s)

        agg_phase(0)
        agg_phase(1)

        if want_deg:
            @pl.when(c == 0)
            def _deg_phase():
                def orow(r, _):
                    ones[r, pl.ds(0, 16)] = jnp.ones((16,), jnp.float32)
                    return 0
                lax.fori_loop(0, CH, orow, 0)
                _zero_z(z[0])
                _zero_acc_slice(z[0], acc, s)
                plsc.subcore_barrier()

                def dchunk(k, _):
                    scidx = s * NCHUNK + k
                    pltpu.sync_copy(comb_all.at[0, scidx], comb0)
                    dcps = [pltpu.async_copy(ones, acc.at[comb0.at[NSUB + j]], ssem[0],
                                             add=True)
                            for j in range(NSUB)]
                    for cp in dcps:
                        cp.wait()
                    return 0
                lax.fori_loop(0, NCHUNK, dchunk, 0)
                plsc.subcore_barrier()
                _copy_out(acc, z[0], lambda r0, nr: out_deg.at[pl.ds(r0, nr)], s)

    return pl.kernel(body, out_type=outs, mesh=mesh, scratch_types=scratch,
                     compiler_params=pltpu.CompilerParams(use_tc_tiling_on_sc=False))


_sc_agg_deg = _build_sc_agg(True)
_sc_agg = _build_sc_agg(False)


def _tc_pre_body(x_ref, a_ref, tout_ref):
    xb = x_ref[...]
    for q in range(4):
        tout_ref[q] = jnp.dot(xb, a_ref[q], preferred_element_type=jnp.float32)


def _tc_mid_body(agg_ref, deg_ref, h_ref, r_ref, b_ref, a_ref, hout_ref, tout_ref):
    agg = jnp.concatenate([agg_ref[0, 0], agg_ref[0, 1],
                           agg_ref[1, 0], agg_ref[1, 1]], axis=-1)
    deg = jnp.maximum(deg_ref[:, 0:1], 1.0)
    pre = (agg / deg
           + jnp.dot(h_ref[...], r_ref[...], preferred_element_type=jnp.float32)
           + b_ref[...])
    hn = jnp.where(pre > 0, pre, jnp.exp(pre) - 1.0)
    hout_ref[...] = hn
    for q in range(4):
        tout_ref[q] = jnp.dot(hn, a_ref[q], preferred_element_type=jnp.float32)


def _tc_final_body(agg_ref, deg_ref, h_ref, r_ref, b_ref, batch_ref,
                   fcw_ref, fcb_ref, out_ref, acc_ref):
    i = pl.program_id(0)

    @pl.when(i == 0)
    def _():
        acc_ref[...] = jnp.zeros((G, 128), jnp.float32)

    agg = jnp.concatenate([agg_ref[0, 0], agg_ref[0, 1],
                           agg_ref[1, 0], agg_ref[1, 1]], axis=-1)
    deg = jnp.maximum(deg_ref[:, 0:1], 1.0)
    pre = (agg / deg
           + jnp.dot(h_ref[...], r_ref[...], preferred_element_type=jnp.float32)
           + b_ref[...])
    h4 = jnp.where(pre > 0, pre, jnp.exp(pre) - 1.0)
    hext = jnp.concatenate([h4, jnp.ones((BLK, F), jnp.float32)], axis=1)
    onehot = (batch_ref[...] ==
              lax.broadcasted_iota(jnp.int32, (BLK, G), 1)).astype(jnp.float32)
    acc_ref[...] += lax.dot_general(onehot, hext, (((0,), (0,)), ((), ())),
                                    preferred_element_type=jnp.float32)

    @pl.when(i == pl.num_programs(0) - 1)
    def _():
        a = acc_ref[...]
        g = a[:, :F] / jnp.maximum(a[:, F:F + 1], 1.0)
        logits = jnp.dot(g, fcw_ref[...], preferred_element_type=jnp.float32) + fcb_ref[...]
        m = jnp.max(logits, axis=1, keepdims=True)
        lse = jnp.log(jnp.sum(jnp.exp(logits - m), axis=1, keepdims=True)) + m
        out_ref[...] = logits - lse


def _tc_pre(x, a):
    fin = x.shape[1]
    return pl.pallas_call(
        _tc_pre_body,
        grid=(NBLK,),
        in_specs=[
            pl.BlockSpec((BLK, fin), lambda i: (i, 0)),
            pl.BlockSpec((4, fin, HF), lambda i: (0, 0, 0)),
        ],
        out_specs=pl.BlockSpec((4, BLK, HF), lambda i: (0, i, 0)),
        out_shape=jax.ShapeDtypeStruct((4, N, HF), jnp.float32),
    )(x, a)


def _tc_mid(agg, deg, h, r, b, a):
    fin = h.shape[1]
    return pl.pallas_call(
        _tc_mid_body,
        grid=(NBLK,),
        in_specs=[
            pl.BlockSpec((2, 2, BLK, QF), lambda i: (0, 0, i, 0)),
            pl.BlockSpec((BLK, QF), lambda i: (i, 0)),
            pl.BlockSpec((BLK, fin), lambda i: (i, 0)),
            pl.BlockSpec((fin, F), lambda i: (0, 0)),
            pl.BlockSpec((1, F), lambda i: (0, 0)),
            pl.BlockSpec((4, F, HF), lambda i: (0, 0, 0)),
        ],
        out_specs=[
            pl.BlockSpec((BLK, F), lambda i: (i, 0)),
            pl.BlockSpec((4, BLK, HF), lambda i: (0, i, 0)),
        ],
        out_shape=[
            jax.ShapeDtypeStruct((N, F), jnp.float32),
            jax.ShapeDtypeStruct((4, N, HF), jnp.float32),
        ],
    )(agg, deg, h, r, b, a)


def _tc_final(agg, deg, h, r, b, batch2, fcw, fcb):
    return pl.pallas_call(
        _tc_final_body,
        grid=(NBLK,),
        in_specs=[
            pl.BlockSpec((2, 2, BLK, QF), lambda i: (0, 0, i, 0)),
            pl.BlockSpec((BLK, QF), lambda i: (i, 0)),
            pl.BlockSpec((BLK, F), lambda i: (i, 0)),
            pl.BlockSpec((F, F), lambda i: (0, 0)),
            pl.BlockSpec((1, F), lambda i: (0, 0)),
            pl.BlockSpec((BLK, 1), lambda i: (i, 0)),
            pl.BlockSpec((F, 6), lambda i: (0, 0)),
            pl.BlockSpec((1, 6), lambda i: (0, 0)),
        ],
        out_specs=pl.BlockSpec((G, 6), lambda i: (0, 0)),
        out_shape=jax.ShapeDtypeStruct((G, 6), jnp.float32),
        scratch_shapes=[pltpu.VMEM((G, 128), jnp.float32)],
    )(agg, deg, h, r, b, batch2, fcw, fcb)


def _amat(w):
    w0 = w[0]
    wd = w[1] - w[0]
    return jnp.stack([
        jnp.concatenate([w0[:, q * QF:(q + 1) * QF], wd[:, q * QF:(q + 1) * QF]],
                        axis=1)
        for q in range(4)
    ])


def kernel(x, edge_index, pseudo, batch, W1, R1, B1, W2, R2, B2, W3, R3, B3,
           W4, R4, B4, fcw, fcb):
    src = edge_index[0].astype(jnp.int32)
    dst = edge_index[1].astype(jnp.int32)
    u = pseudo[:, 0]
    npad = EPAD - E
    src = jnp.concatenate([src, jnp.zeros((npad,), jnp.int32)])
    dst = jnp.concatenate([dst, jnp.full((npad,), N, jnp.int32)])
    u = jnp.concatenate([u, jnp.zeros((npad,), jnp.float32)])
    nsc = EPAD // SUP
    dstm = dst.reshape(nsc, NSUB, CH)
    um = jax.lax.bitcast_convert_type(u, jnp.int32).reshape(nsc, NSUB, CH)
    comb_all = jnp.stack([
        jnp.concatenate([(src + qq * N).reshape(nsc, NSUB, CH), dstm, um], axis=1)
        for qq in range(4)
    ])
    comb_all = jnp.concatenate(
        [comb_all, jnp.zeros((4, 2, 3 * NSUB, CH), jnp.int32)], axis=1)
    batch2 = batch.astype(jnp.int32).reshape(N, 1)

    t = _tc_pre(x, _amat(W1))
    agg, deg16 = _sc_agg_deg(t.reshape(4 * N, 2 * QF), comb_all)
    agg, deg16 = agg[:, :, :N], deg16[:N]
    h, t = _tc_mid(agg, deg16, x, R1, B1.reshape(1, F), _amat(W2))
    agg = _sc_agg(t.reshape(4 * N, 2 * QF), comb_all)[:, :, :N]
    h, t = _tc_mid(agg, deg16, h, R2, B2.reshape(1, F), _amat(W3))
    agg = _sc_agg(t.reshape(4 * N, 2 * QF), comb_all)[:, :, :N]
    h, t = _tc_mid(agg, deg16, h, R3, B3.reshape(1, F), _amat(W4))
    agg = _sc_agg(t.reshape(4 * N, 2 * QF), comb_all)[:, :, :N]
    return _tc_final(agg, deg16, h, R4, B4.reshape(1, F), batch2, fcw,
                     fcb.reshape(1, 6))
